# baseline (speedup 1.0000x reference)
import functools

import jax
import jax.numpy as jnp
from jax.experimental import pallas as pl
from jax.experimental.pallas import tpu as pltpu

EMBED = 4096
NEXP = 64
K = 8
BT = 256


def _gate_body(x_ref, w_ref, logits_ref, dense_ref, tw_ref, ti_ref):
    x = x_ref[...]
    w = w_ref[...]
    logits = jax.lax.dot_general(
        x, w, (((1,), (1,)), ((), ())), preferred_element_type=jnp.float32
    )
    logits_ref[...] = logits

    lane = jax.lax.broadcasted_iota(jnp.int32, logits.shape, 1)
    work = logits
    vals = []
    for k in range(K):
        m = jnp.max(work, axis=1, keepdims=True)
        idx = jnp.min(jnp.where(work == m, lane, NEXP), axis=1, keepdims=True)
        vals.append(m)
        ti_ref[:, k:k + 1] = idx
        work = jnp.where(lane == idx, -jnp.inf, work)

    m0 = vals[0]
    e = jnp.exp(logits - m0)
    dense_ref[...] = e / jnp.sum(e, axis=1, keepdims=True)

    te = [jnp.exp(v - m0) for v in vals]
    tsum = functools.reduce(jnp.add, te)
    for k in range(K):
        tw_ref[:, k:k + 1] = te[k] / tsum


def kernel(x, W):
    n_tokens = x.shape[0]
    grid = (n_tokens // BT,)
    out_shapes = (
        jax.ShapeDtypeStruct((n_tokens, NEXP), jnp.float32),
        jax.ShapeDtypeStruct((n_tokens, NEXP), jnp.float32),
        jax.ShapeDtypeStruct((n_tokens, K), jnp.float32),
        jax.ShapeDtypeStruct((n_tokens, K), jnp.int32),
    )
    out_specs = (
        pl.BlockSpec((BT, NEXP), lambda i: (i, 0)),
        pl.BlockSpec((BT, NEXP), lambda i: (i, 0)),
        pl.BlockSpec((BT, K), lambda i: (i, 0)),
        pl.BlockSpec((BT, K), lambda i: (i, 0)),
    )
    in_specs = [
        pl.BlockSpec((BT, EMBED), lambda i: (i, 0)),
        pl.BlockSpec((NEXP, EMBED), lambda i: (0, 0)),
    ]
    return pl.pallas_call(
        _gate_body,
        grid=grid,
        in_specs=in_specs,
        out_specs=out_specs,
        out_shape=out_shapes,
        compiler_params=pltpu.CompilerParams(
            dimension_semantics=("arbitrary",),
        ),
    )(x, W)


# speedup vs baseline: 1.3798x; 1.3798x over previous
import functools

import jax
import jax.numpy as jnp
from jax.experimental import pallas as pl
from jax.experimental.pallas import tpu as pltpu

EMBED = 4096
NEXP = 64
K = 8
BT = 256


def _gate_body(x_ref, w_ref, logits_ref, dense_ref, tw_ref, ti_ref):
    x = x_ref[...]
    w = w_ref[...]
    logits_t = jax.lax.dot_general(
        w, x, (((1,), (1,)), ((), ())), preferred_element_type=jnp.float32
    )

    sub = jax.lax.broadcasted_iota(jnp.int32, logits_t.shape, 0)
    work = logits_t
    vals = []
    idxs = []
    for k in range(K):
        m = jnp.max(work, axis=0, keepdims=True)
        idx = jnp.min(jnp.where(work == m, sub, NEXP), axis=0, keepdims=True)
        vals.append(m)
        idxs.append(idx)
        work = jnp.where(sub == idx, -jnp.inf, work)

    m0 = vals[0]
    e_t = jnp.exp(logits_t - m0)
    dense_t = e_t / jnp.sum(e_t, axis=0, keepdims=True)

    tvals_t = jnp.concatenate(vals, axis=0)
    tidx_t = jnp.concatenate(idxs, axis=0)
    te_t = jnp.exp(tvals_t - m0)
    tw_t = te_t / jnp.sum(te_t, axis=0, keepdims=True)

    logits_ref[...] = logits_t.T
    dense_ref[...] = dense_t.T
    tw_ref[...] = tw_t.T
    ti_ref[...] = tidx_t.T


def kernel(x, W):
    n_tokens = x.shape[0]
    grid = (n_tokens // BT,)
    out_shapes = (
        jax.ShapeDtypeStruct((n_tokens, NEXP), jnp.float32),
        jax.ShapeDtypeStruct((n_tokens, NEXP), jnp.float32),
        jax.ShapeDtypeStruct((n_tokens, K), jnp.float32),
        jax.ShapeDtypeStruct((n_tokens, K), jnp.int32),
    )
    out_specs = (
        pl.BlockSpec((BT, NEXP), lambda i: (i, 0)),
        pl.BlockSpec((BT, NEXP), lambda i: (i, 0)),
        pl.BlockSpec((BT, K), lambda i: (i, 0)),
        pl.BlockSpec((BT, K), lambda i: (i, 0)),
    )
    in_specs = [
        pl.BlockSpec((BT, EMBED), lambda i: (i, 0)),
        pl.BlockSpec((NEXP, EMBED), lambda i: (0, 0)),
    ]
    return pl.pallas_call(
        _gate_body,
        grid=grid,
        in_specs=in_specs,
        out_specs=out_specs,
        out_shape=out_shapes,
        compiler_params=pltpu.CompilerParams(
            dimension_semantics=("arbitrary",),
        ),
    )(x, W)


# speedup vs baseline: 1.5853x; 1.1489x over previous
import functools

import jax
import jax.numpy as jnp
from jax.experimental import pallas as pl
from jax.experimental.pallas import tpu as pltpu

EMBED = 4096
NEXP = 64
K = 8
BT = 512


def _gate_body(x_ref, w_ref, logits_ref, dense_ref, tw_ref, ti_ref):
    x = x_ref[...]
    w = w_ref[...]
    logits_t = jax.lax.dot_general(
        w, x, (((1,), (1,)), ((), ())), preferred_element_type=jnp.float32
    )

    sub = jax.lax.broadcasted_iota(jnp.int32, logits_t.shape, 0)
    work = logits_t
    vals = []
    idxs = []
    for k in range(K):
        m = jnp.max(work, axis=0, keepdims=True)
        idx = jnp.min(jnp.where(work == m, sub, NEXP), axis=0, keepdims=True)
        vals.append(m)
        idxs.append(idx)
        work = jnp.where(sub == idx, -jnp.inf, work)

    m0 = vals[0]
    e_t = jnp.exp(logits_t - m0)
    dense_t = e_t / jnp.sum(e_t, axis=0, keepdims=True)

    tvals_t = jnp.concatenate(vals, axis=0)
    tidx_t = jnp.concatenate(idxs, axis=0)
    te_t = jnp.exp(tvals_t - m0)
    tw_t = te_t / jnp.sum(te_t, axis=0, keepdims=True)

    logits_ref[...] = logits_t.T
    dense_ref[...] = dense_t.T
    tw_ref[...] = tw_t.T
    ti_ref[...] = tidx_t.T


def kernel(x, W):
    n_tokens = x.shape[0]
    grid = (n_tokens // BT,)
    out_shapes = (
        jax.ShapeDtypeStruct((n_tokens, NEXP), jnp.float32),
        jax.ShapeDtypeStruct((n_tokens, NEXP), jnp.float32),
        jax.ShapeDtypeStruct((n_tokens, K), jnp.float32),
        jax.ShapeDtypeStruct((n_tokens, K), jnp.int32),
    )
    out_specs = (
        pl.BlockSpec((BT, NEXP), lambda i: (i, 0)),
        pl.BlockSpec((BT, NEXP), lambda i: (i, 0)),
        pl.BlockSpec((BT, K), lambda i: (i, 0)),
        pl.BlockSpec((BT, K), lambda i: (i, 0)),
    )
    in_specs = [
        pl.BlockSpec((BT, EMBED), lambda i: (i, 0)),
        pl.BlockSpec((NEXP, EMBED), lambda i: (0, 0)),
    ]
    return pl.pallas_call(
        _gate_body,
        grid=grid,
        in_specs=in_specs,
        out_specs=out_specs,
        out_shape=out_shapes,
        compiler_params=pltpu.CompilerParams(
            dimension_semantics=("arbitrary",),
        ),
    )(x, W)


# speedup vs baseline: 1.6725x; 1.0550x over previous
import functools

import jax
import jax.numpy as jnp
from jax.experimental import pallas as pl
from jax.experimental.pallas import tpu as pltpu

EMBED = 4096
NEXP = 64
K = 8
BT = 1024


def _gate_body(x_ref, w_ref, logits_ref, dense_ref, tw_ref, ti_ref):
    x = x_ref[...]
    w = w_ref[...]
    logits_t = jax.lax.dot_general(
        w, x, (((1,), (1,)), ((), ())), preferred_element_type=jnp.float32
    )

    sub = jax.lax.broadcasted_iota(jnp.int32, logits_t.shape, 0)
    work = logits_t
    vals = []
    idxs = []
    for k in range(K):
        m = jnp.max(work, axis=0, keepdims=True)
        idx = jnp.min(jnp.where(work == m, sub, NEXP), axis=0, keepdims=True)
        vals.append(m)
        idxs.append(idx)
        work = jnp.where(sub == idx, -jnp.inf, work)

    m0 = vals[0]
    e_t = jnp.exp(logits_t - m0)
    dense_t = e_t / jnp.sum(e_t, axis=0, keepdims=True)

    tvals_t = jnp.concatenate(vals, axis=0)
    tidx_t = jnp.concatenate(idxs, axis=0)
    te_t = jnp.exp(tvals_t - m0)
    tw_t = te_t / jnp.sum(te_t, axis=0, keepdims=True)

    logits_ref[...] = logits_t.T
    dense_ref[...] = dense_t.T
    tw_ref[...] = tw_t.T
    ti_ref[...] = tidx_t.T


def kernel(x, W):
    n_tokens = x.shape[0]
    grid = (n_tokens // BT,)
    out_shapes = (
        jax.ShapeDtypeStruct((n_tokens, NEXP), jnp.float32),
        jax.ShapeDtypeStruct((n_tokens, NEXP), jnp.float32),
        jax.ShapeDtypeStruct((n_tokens, K), jnp.float32),
        jax.ShapeDtypeStruct((n_tokens, K), jnp.int32),
    )
    out_specs = (
        pl.BlockSpec((BT, NEXP), lambda i: (i, 0)),
        pl.BlockSpec((BT, NEXP), lambda i: (i, 0)),
        pl.BlockSpec((BT, K), lambda i: (i, 0)),
        pl.BlockSpec((BT, K), lambda i: (i, 0)),
    )
    in_specs = [
        pl.BlockSpec((BT, EMBED), lambda i: (i, 0)),
        pl.BlockSpec((NEXP, EMBED), lambda i: (0, 0)),
    ]
    return pl.pallas_call(
        _gate_body,
        grid=grid,
        in_specs=in_specs,
        out_specs=out_specs,
        out_shape=out_shapes,
        compiler_params=pltpu.CompilerParams(
            dimension_semantics=("arbitrary",),
        ),
    )(x, W)
